# Initial kernel scaffold; baseline (speedup 1.0000x reference)
import jax
import jax.numpy as jnp
from jax.experimental import pallas as pl


def kernel(x, edge_index, edge_attr, W1, b1, W2, b2):
    raise NotImplementedError("write your pallas kernel here")


# speedup vs baseline: 2.5001x; 2.5001x over previous
import functools

import jax
import jax.numpy as jnp
from jax import lax
from jax.experimental import pallas as pl
from jax.experimental.pallas import tpu as pltpu
from jax.experimental.pallas import tpu_sc as plsc

N_NODES = 10000
N_EDGES = 320000
D_FEAT = 128
D_HALF = 64
D_EDGE = 16

NW = 32
EPW = N_EDGES // NW
CH = 80
NCH = EPW // CH
N_PAD = 10240
NPT = N_PAD // 16


def _rsqrt16(a):
    i = plsc.bitcast(a, jnp.int32)
    i = jnp.int32(0x5F3759DF) - (i >> 1)
    y = plsc.bitcast(i, jnp.float32)
    for _ in range(3):
        y = y * (jnp.float32(1.5) - jnp.float32(0.5) * a * y * y)
    return y


def _sc_aggregate_body(x0_hbm, x1_hbm, sqn_hbm, row_hbm, col_hbm, ea_hbm,
                       sqna_hbm,
                       aggae_out, aggx0_out, aggx1_out,
                       sqn_v, row_v, col_v, sqna_v, ea_v, rows_v, aev_v,
                       w_v, zx_v, agg_sh, sem):
    cid = lax.axis_index("c")
    sid = lax.axis_index("s")
    wid = cid * 16 + sid

    pltpu.sync_copy(sqn_hbm, sqn_v)
    pltpu.sync_copy(row_hbm.at[wid], row_v)
    pltpu.sync_copy(col_hbm.at[wid], col_v)
    pltpu.sync_copy(sqna_hbm.at[wid], sqna_v)

    zeros16 = jnp.zeros((16,), jnp.float32)

    def zero_row(i, c):
        for d in range(4):
            zx_v[i, pl.ds(d * 16, 16)] = zeros16
            aev_v[i % CH, pl.ds(d * 16, 16)] = zeros16
        return c

    lax.fori_loop(0, 128, zero_row, 0)

    onehot = jnp.where(lax.iota(jnp.int32, 16) == 0,
                       jnp.float32(1.0), jnp.float32(0.0))
    zbase = sid * NPT

    for p, (xt_hbm, out_hbm) in enumerate(
            ((None, aggae_out), (x0_hbm, aggx0_out), (x1_hbm, aggx1_out))):
        for t in range(5):
            pltpu.sync_copy(zx_v, agg_sh.at[pl.ds(zbase + t * 128, 128)])
        plsc.subcore_barrier()

        def chunk(j, c, p=p, xt_hbm=xt_hbm):
            if p == 0:
                pltpu.sync_copy(ea_hbm.at[wid, j], ea_v)
                for k in range(CH // 16):
                    ridx = row_v[j, pl.ds(k * 16, 16)]
                    sq = (plsc.load_gather(sqn_v, [ridx])
                          + sqna_v[j, pl.ds(k * 16, 16)])
                    y = _rsqrt16(jnp.maximum(sq, jnp.float32(1e-30)))
                    w = jnp.where(sq > jnp.float32(1e-24), y,
                                  jnp.float32(1e12))
                    w_v[j, pl.ds(k * 16, 16)] = w
            else:
                pltpu.async_copy(xt_hbm.at[row_v.at[j]], rows_v, sem).wait()

            def per_edge(i, cc):
                w = w_v[j, pl.ds(i, 16)][0]
                if p == 0:
                    aev_v[i, pl.ds(0, 16)] = ea_v[i, :] * w
                    aev_v[i, pl.ds(16, 16)] = onehot
                else:
                    for d in range(4):
                        rows_v[i, pl.ds(d * 16, 16)] = \
                            rows_v[i, pl.ds(d * 16, 16)] * w
                return cc

            lax.fori_loop(0, CH, per_edge, 0)
            src = aev_v if p == 0 else rows_v
            pltpu.sync_copy(src, agg_sh.at[col_v.at[j]], add=True)
            return c

        lax.fori_loop(0, NCH, chunk, 0)
        plsc.subcore_barrier()

        pltpu.sync_copy(agg_sh.at[pl.ds(zbase, NPT)],
                        out_hbm.at[cid, pl.ds(zbase, NPT)])


_SC_MESH = plsc.VectorSubcoreMesh(core_axis_name="c", subcore_axis_name="s")

_sc_aggregate = pl.kernel(
    _sc_aggregate_body,
    out_type=(
        jax.ShapeDtypeStruct((2, N_PAD, D_HALF), jnp.float32),
        jax.ShapeDtypeStruct((2, N_PAD, D_HALF), jnp.float32),
        jax.ShapeDtypeStruct((2, N_PAD, D_HALF), jnp.float32),
    ),
    mesh=_SC_MESH,
    scratch_types=[
        pltpu.VMEM((N_NODES,), jnp.float32),
        pltpu.VMEM((NCH, CH), jnp.int32),
        pltpu.VMEM((NCH, CH), jnp.int32),
        pltpu.VMEM((NCH, CH), jnp.float32),
        pltpu.VMEM((CH, D_EDGE), jnp.float32),
        pltpu.VMEM((CH, D_HALF), jnp.float32),
        pltpu.VMEM((CH, D_HALF), jnp.float32),
        pltpu.VMEM((NCH, CH + 16), jnp.float32),
        pltpu.VMEM((128, D_HALF), jnp.float32),
        pltpu.VMEM_SHARED((N_PAD, D_HALF), jnp.float32),
        pltpu.SemaphoreType.DMA,
    ],
    compiler_params=pltpu.CompilerParams(needs_layout_passes=False,
                                         use_tc_tiling_on_sc=False),
    name="sage_sc_aggregate",
)


def _sq_body(x_ref, o_ref):
    v = x_ref[...]
    o_ref[...] = jnp.sum(v * v, axis=1, keepdims=True)


def _row_sqnorm(x, blk):
    rows, d = x.shape
    return pl.pallas_call(
        _sq_body,
        grid=(rows // blk,),
        in_specs=[pl.BlockSpec((blk, d), lambda i: (i, 0))],
        out_specs=pl.BlockSpec((blk, 1), lambda i: (i, 0)),
        out_shape=jax.ShapeDtypeStruct((rows, 1), jnp.float32),
    )(x)


def _post_body(ae_ref, ax0_ref, ax1_ref, wx0_ref, wx1_ref, we_ref, b_ref,
               o_ref, *, act):
    ae = ae_ref[0] + ae_ref[1]
    ax0 = ax0_ref[0] + ax0_ref[1]
    ax1 = ax1_ref[0] + ax1_ref[1]
    cnt = ae[:, 16:17]
    inv = jnp.float32(1.0) / jnp.maximum(cnt, jnp.float32(1.0))
    h = (lax.dot(ax0, wx0_ref[...], preferred_element_type=jnp.float32)
         + lax.dot(ax1, wx1_ref[...], preferred_element_type=jnp.float32)
         + lax.dot(ae[:, :16], we_ref[...], preferred_element_type=jnp.float32))
    h = h * inv + b_ref[...]
    n = jnp.sqrt(jnp.sum(h * h, axis=1, keepdims=True))
    h = h / jnp.maximum(n, jnp.float32(1e-12))
    o_ref[...] = act(h)


def _post(ae, ax0, ax1, wx0, wx1, we, b, act, blk=1000):
    body = functools.partial(_post_body, act=act)
    return pl.pallas_call(
        body,
        grid=(N_NODES // blk,),
        in_specs=[
            pl.BlockSpec((2, blk, D_HALF), lambda i: (0, i, 0)),
            pl.BlockSpec((2, blk, D_HALF), lambda i: (0, i, 0)),
            pl.BlockSpec((2, blk, D_HALF), lambda i: (0, i, 0)),
            pl.BlockSpec((D_HALF, D_FEAT), lambda i: (0, 0)),
            pl.BlockSpec((D_HALF, D_FEAT), lambda i: (0, 0)),
            pl.BlockSpec((D_EDGE, D_FEAT), lambda i: (0, 0)),
            pl.BlockSpec((1, D_FEAT), lambda i: (0, 0)),
        ],
        out_specs=pl.BlockSpec((blk, D_FEAT), lambda i: (i, 0)),
        out_shape=jax.ShapeDtypeStruct((N_NODES, D_FEAT), jnp.float32),
    )(ae, ax0, ax1, wx0, wx1, we, b)


def _layer(xfeat, sqn, row, col, ea_r, sqna, W, b, act):
    ae, ax0, ax1 = _sc_aggregate(xfeat[:, :D_HALF], xfeat[:, D_HALF:],
                                 sqn, row, col, ea_r, sqna)
    return _post(ae[:, :N_NODES], ax0[:, :N_NODES], ax1[:, :N_NODES],
                 W[:D_HALF], W[D_HALF:D_FEAT], W[D_FEAT:],
                 b.reshape(1, D_FEAT), act)


def kernel(x, edge_index, edge_attr, W1, b1, W2, b2):
    ei = edge_index.astype(jnp.int32)
    row = ei[0].reshape(NW, NCH, CH)
    col = ei[1].reshape(NW, NCH, CH)
    ea_r = edge_attr.reshape(NW, NCH, CH, D_EDGE)

    sqna = _row_sqnorm(edge_attr, blk=8000).reshape(NW, NCH, CH)
    sqnx = _row_sqnorm(x, blk=1000).reshape(N_NODES)

    h1 = _layer(x, sqnx, row, col, ea_r, sqna, W1, b1,
                lambda h: jnp.maximum(h, 0.0))
    sqh1 = _row_sqnorm(h1, blk=1000).reshape(N_NODES)
    out = _layer(h1, sqh1, row, col, ea_r, sqna, W2, b2,
                 lambda h: jax.nn.sigmoid(h))
    return out


# speedup vs baseline: 3.9852x; 1.5940x over previous
import functools

import jax
import jax.numpy as jnp
from jax import lax
from jax.experimental import pallas as pl
from jax.experimental.pallas import tpu as pltpu
from jax.experimental.pallas import tpu_sc as plsc

N_NODES = 10000
N_EDGES = 320000
D_FEAT = 128
D_HALF = 64
D_EDGE = 16

NW = 32
EPW = N_EDGES // NW
CH = 80
NCH = EPW // CH
N_PAD = 10240
NPT = N_PAD // 16


def _rsqrt16(a):
    i = plsc.bitcast(a, jnp.int32)
    i = jnp.int32(0x5F3759DF) - (i >> 1)
    y = plsc.bitcast(i, jnp.float32)
    for _ in range(3):
        y = y * (jnp.float32(1.5) - jnp.float32(0.5) * a * y * y)
    return y


NBUF = 5


def _sc_aggregate_body(x0_hbm, x1_hbm, sqn_hbm, row_hbm, col_hbm, ea_hbm,
                       sqna_hbm,
                       aggae_out, aggx0_out, aggx1_out,
                       sqn_v, row_v, col_v, sqna_v, ea_v, bufs, w_v, zx_v,
                       agg_sh, gsem, ssem):
    cid = lax.axis_index("c")
    sid = lax.axis_index("s")
    wid = cid * 16 + sid

    pltpu.sync_copy(sqn_hbm, sqn_v)
    pltpu.sync_copy(row_hbm.at[wid], row_v)
    pltpu.sync_copy(col_hbm.at[wid], col_v)
    pltpu.sync_copy(sqna_hbm.at[wid], sqna_v)

    zeros16 = jnp.zeros((16,), jnp.float32)

    def zero_row(i, c):
        for d in range(4):
            zx_v[i, pl.ds(d * 16, 16)] = zeros16
        return c

    lax.fori_loop(0, 128, zero_row, 0)

    onehot = jnp.where(lax.iota(jnp.int32, 16) == 0,
                       jnp.float32(1.0), jnp.float32(0.0))
    zbase = sid * NPT

    def wait_scatter(j):
        pltpu.make_async_copy(bufs.at[0], agg_sh.at[col_v.at[j]],
                              ssem).wait()

    for p, (xt_hbm, out_hbm) in enumerate(
            ((None, aggae_out), (x0_hbm, aggx0_out), (x1_hbm, aggx1_out))):
        for t in range(5):
            pltpu.sync_copy(zx_v, agg_sh.at[pl.ds(zbase + t * 128, 128)])
        plsc.subcore_barrier()

        if p > 0:
            pltpu.async_copy(xt_hbm.at[row_v.at[0]], bufs.at[0], gsem)
            pltpu.async_copy(xt_hbm.at[row_v.at[1]], bufs.at[1], gsem)

        def outer(j5, c, p=p, xt_hbm=xt_hbm):
            for r in range(NBUF):
                j = j5 * NBUF + r
                buf = bufs.at[r]
                pl.when(j >= 3)(lambda: wait_scatter(j - 3))
                if p > 0:
                    def _ahead(j=j, r=r, xt_hbm=xt_hbm):
                        pltpu.async_copy(xt_hbm.at[row_v.at[j + 2]],
                                         bufs.at[(r + 2) % NBUF], gsem)
                    pl.when(j + 2 < NCH)(_ahead)
                    pltpu.make_async_copy(xt_hbm.at[row_v.at[j]], buf,
                                          gsem).wait()
                else:
                    pltpu.sync_copy(ea_hbm.at[wid, j], ea_v)
                    for k in range(CH // 16):
                        ridx = row_v[j, pl.ds(k * 16, 16)]
                        sq = (plsc.load_gather(sqn_v, [ridx])
                              + sqna_v[j, pl.ds(k * 16, 16)])
                        y = _rsqrt16(jnp.maximum(sq, jnp.float32(1e-30)))
                        w = jnp.where(sq > jnp.float32(1e-24), y,
                                      jnp.float32(1e12))
                        w_v[j, pl.ds(k * 16, 16)] = w

                def per_edge(i, cc, buf=buf, p=p):
                    w = w_v[j, pl.ds(i, 16)][0]
                    if p == 0:
                        buf[i, pl.ds(0, 16)] = ea_v[i, :] * w
                        buf[i, pl.ds(16, 16)] = onehot
                        buf[i, pl.ds(32, 16)] = zeros16
                        buf[i, pl.ds(48, 16)] = zeros16
                    else:
                        for d in range(4):
                            buf[i, pl.ds(d * 16, 16)] = \
                                buf[i, pl.ds(d * 16, 16)] * w
                    return cc

                lax.fori_loop(0, CH, per_edge, 0, unroll=2)
                pltpu.async_copy(buf, agg_sh.at[col_v.at[j]], ssem, add=True)
            return c

        lax.fori_loop(0, NCH // NBUF, outer, 0)
        for jt in (NCH - 3, NCH - 2, NCH - 1):
            wait_scatter(jt)
        plsc.subcore_barrier()

        pltpu.sync_copy(agg_sh.at[pl.ds(zbase, NPT)],
                        out_hbm.at[cid, pl.ds(zbase, NPT)])


_SC_MESH = plsc.VectorSubcoreMesh(core_axis_name="c", subcore_axis_name="s")

_sc_aggregate = pl.kernel(
    _sc_aggregate_body,
    out_type=(
        jax.ShapeDtypeStruct((2, N_PAD, D_HALF), jnp.float32),
        jax.ShapeDtypeStruct((2, N_PAD, D_HALF), jnp.float32),
        jax.ShapeDtypeStruct((2, N_PAD, D_HALF), jnp.float32),
    ),
    mesh=_SC_MESH,
    scratch_types=[
        pltpu.VMEM((N_NODES,), jnp.float32),
        pltpu.VMEM((NCH, CH), jnp.int32),
        pltpu.VMEM((NCH, CH), jnp.int32),
        pltpu.VMEM((NCH, CH), jnp.float32),
        pltpu.VMEM((CH, D_EDGE), jnp.float32),
        pltpu.VMEM((NBUF, CH, D_HALF), jnp.float32),
        pltpu.VMEM((NCH, CH + 16), jnp.float32),
        pltpu.VMEM((128, D_HALF), jnp.float32),
        pltpu.VMEM_SHARED((N_PAD, D_HALF), jnp.float32),
        pltpu.SemaphoreType.DMA,
        pltpu.SemaphoreType.DMA,
    ],
    compiler_params=pltpu.CompilerParams(needs_layout_passes=False,
                                         use_tc_tiling_on_sc=False),
    name="sage_sc_aggregate",
)


def _sq_body(x_ref, o_ref):
    v = x_ref[...]
    o_ref[...] = jnp.sum(v * v, axis=1, keepdims=True)


def _row_sqnorm(x, blk):
    rows, d = x.shape
    return pl.pallas_call(
        _sq_body,
        grid=(rows // blk,),
        in_specs=[pl.BlockSpec((blk, d), lambda i: (i, 0))],
        out_specs=pl.BlockSpec((blk, 1), lambda i: (i, 0)),
        out_shape=jax.ShapeDtypeStruct((rows, 1), jnp.float32),
    )(x)


def _post_body(ae_ref, ax0_ref, ax1_ref, wx0_ref, wx1_ref, we_ref, b_ref,
               o_ref, sq_ref, *, act):
    ae = ae_ref[0] + ae_ref[1]
    ax0 = ax0_ref[0] + ax0_ref[1]
    ax1 = ax1_ref[0] + ax1_ref[1]
    cnt = ae[:, 16:17]
    inv = jnp.float32(1.0) / jnp.maximum(cnt, jnp.float32(1.0))
    h = (lax.dot(ax0, wx0_ref[...], preferred_element_type=jnp.float32)
         + lax.dot(ax1, wx1_ref[...], preferred_element_type=jnp.float32)
         + lax.dot(ae[:, :16], we_ref[...], preferred_element_type=jnp.float32))
    h = h * inv + b_ref[...]
    n = jnp.sqrt(jnp.sum(h * h, axis=1, keepdims=True))
    h = act(h / jnp.maximum(n, jnp.float32(1e-12)))
    o_ref[...] = h
    sq_ref[...] = jnp.sum(h * h, axis=1, keepdims=True)


def _post(ae, ax0, ax1, wx0, wx1, we, b, act, blk=1000):
    body = functools.partial(_post_body, act=act)
    return pl.pallas_call(
        body,
        grid=(N_NODES // blk,),
        in_specs=[
            pl.BlockSpec((2, blk, D_HALF), lambda i: (0, i, 0)),
            pl.BlockSpec((2, blk, D_HALF), lambda i: (0, i, 0)),
            pl.BlockSpec((2, blk, D_HALF), lambda i: (0, i, 0)),
            pl.BlockSpec((D_HALF, D_FEAT), lambda i: (0, 0)),
            pl.BlockSpec((D_HALF, D_FEAT), lambda i: (0, 0)),
            pl.BlockSpec((D_EDGE, D_FEAT), lambda i: (0, 0)),
            pl.BlockSpec((1, D_FEAT), lambda i: (0, 0)),
        ],
        out_specs=[
            pl.BlockSpec((blk, D_FEAT), lambda i: (i, 0)),
            pl.BlockSpec((blk, 1), lambda i: (i, 0)),
        ],
        out_shape=[
            jax.ShapeDtypeStruct((N_NODES, D_FEAT), jnp.float32),
            jax.ShapeDtypeStruct((N_NODES, 1), jnp.float32),
        ],
    )(ae, ax0, ax1, wx0, wx1, we, b)


def _layer(xfeat, sqn, row, col, ea_r, sqna, W, b, act):
    ae, ax0, ax1 = _sc_aggregate(xfeat[:, :D_HALF], xfeat[:, D_HALF:],
                                 sqn, row, col, ea_r, sqna)
    return _post(ae[:, :N_NODES], ax0[:, :N_NODES], ax1[:, :N_NODES],
                 W[:D_HALF], W[D_HALF:D_FEAT], W[D_FEAT:],
                 b.reshape(1, D_FEAT), act)


def kernel(x, edge_index, edge_attr, W1, b1, W2, b2):
    ei = edge_index.astype(jnp.int32)
    row = ei[0].reshape(NW, NCH, CH)
    col = ei[1].reshape(NW, NCH, CH)
    ea_r = edge_attr.reshape(NW, NCH, CH, D_EDGE)

    sqna = _row_sqnorm(edge_attr, blk=8000).reshape(NW, NCH, CH)
    sqnx = _row_sqnorm(x, blk=1000).reshape(N_NODES)

    h1, sqh1 = _layer(x, sqnx, row, col, ea_r, sqna, W1, b1,
                      lambda h: jnp.maximum(h, 0.0))
    out, _ = _layer(h1, sqh1.reshape(N_NODES), row, col, ea_r, sqna, W2, b2,
                    lambda h: jax.nn.sigmoid(h))
    return out
